# baseline (speedup 1.0000x reference)
import functools

import jax
import jax.numpy as jnp
from jax import lax
from jax.experimental import pallas as pl
from jax.experimental.pallas import tpu as pltpu
from jax.experimental.pallas import tpu_sc as plsc

N_CAT = 26
VOCAB1 = 100001
EMB = 32
BATCH = 16384
NROWS = BATCH * N_CAT

NC = 2
NS = 16
L = 16
NW = NC * NS
RPW = NROWS // NW
CH = 128
NCH = RPW // CH
NBUF = 4


def _sc_gather_body(x_hbm, tbl_hbm, out_hbm, xv, idxv,
                    b0, b1, b2, b3, s0, s1, s2, s3):
    bufs = [b0, b1, b2, b3]
    sems = [s0, s1, s2, s3]
    wid = lax.axis_index("s") * NC + lax.axis_index("c")
    base = pl.multiple_of(wid * RPW, CH)

    pltpu.sync_copy(x_hbm.at[pl.ds(base, RPW)], xv)

    def ibody(c, carry):
        lanes = lax.iota(jnp.int32, L) + (base + c * L)
        f = lanes % N_CAT
        idxv[pl.ds(c * L, L)] = xv[pl.ds(c * L, L)] + 1 + f * VOCAB1
        return carry

    lax.fori_loop(0, RPW // L, ibody, 0)

    def start(j, b):
        off = pl.multiple_of(j * CH, CH)
        pltpu.async_copy(tbl_hbm.at[idxv.at[pl.ds(off, CH)]], bufs[b], sems[b])

    for b in range(NBUF):
        start(b, b)

    def gbody(g, carry):
        for b in range(NBUF):
            j = g * NBUF + b
            pltpu.make_async_copy(tbl_hbm.at[pl.ds(0, CH)], bufs[b],
                                  sems[b]).wait()
            dst = pl.multiple_of(base + j * CH, CH)
            pltpu.sync_copy(bufs[b], out_hbm.at[pl.ds(dst, CH)])
            nxt = j + NBUF

            @pl.when(nxt < NCH)
            def _():
                start(nxt, b)
        return carry

    lax.fori_loop(0, NCH // NBUF, gbody, 0)


def _sc_gather(x_flat, tbl_flat):
    mesh = plsc.VectorSubcoreMesh(core_axis_name="c", subcore_axis_name="s",
                                  num_cores=NC, num_subcores=NS)
    return pl.kernel(
        _sc_gather_body,
        out_type=jax.ShapeDtypeStruct((NROWS, EMB), jnp.float32),
        mesh=mesh,
        scratch_types=[
            pltpu.VMEM((RPW,), jnp.int32),
            pltpu.VMEM((RPW,), jnp.int32),
        ] + [pltpu.VMEM((CH, EMB), jnp.float32)] * NBUF
          + [pltpu.SemaphoreType.DMA] * NBUF,
        compiler_params=pltpu.CompilerParams(use_tc_tiling_on_sc=False),
        name="sc_embed_gather",
    )(x_flat, tbl_flat)


RBLK = 2048
NBLK = -(-VOCAB1 // RBLK)


def _maxabs_body(tbl_ref, out_ref):
    nb = pl.program_id(1)

    @pl.when(nb == 0)
    def _():
        out_ref[...] = jnp.zeros_like(out_ref)

    xb = tbl_ref[0]
    rows = nb * RBLK + lax.broadcasted_iota(jnp.int32, (RBLK, EMB), 0)
    m = jnp.max(jnp.where(rows < VOCAB1, jnp.abs(xb), 0.0))
    out_ref[...] = jnp.maximum(out_ref[...], m)


def _maxabs(tables):
    return pl.pallas_call(
        _maxabs_body,
        grid=(N_CAT, NBLK),
        in_specs=[pl.BlockSpec((1, RBLK, EMB), lambda f, nb: (f, nb, 0))],
        out_specs=pl.BlockSpec((1, 8, 128), lambda f, nb: (f, 0, 0)),
        out_shape=jax.ShapeDtypeStruct((N_CAT, 8, 128), jnp.float32),
    )(tables)


RB = 512
FDIM = N_CAT * EMB


def _tanh_body(g_ref, s_ref, o_ref):
    o_ref[...] = jnp.tanh(0.2 * g_ref[...] / s_ref[...])


def _tanh_scale(g2, scale_row):
    return pl.pallas_call(
        _tanh_body,
        grid=(BATCH // RB,),
        in_specs=[pl.BlockSpec((RB, FDIM), lambda i: (i, 0)),
                  pl.BlockSpec((1, FDIM), lambda i: (0, 0))],
        out_specs=pl.BlockSpec((RB, FDIM), lambda i: (i, 0)),
        out_shape=jax.ShapeDtypeStruct((BATCH, FDIM), jnp.float32),
    )(g2, scale_row)


def kernel(x, tables):
    x_flat = x.reshape(NROWS).astype(jnp.int32)
    tbl_flat = tables.reshape(N_CAT * VOCAB1, EMB)

    gathered = _sc_gather(x_flat, tbl_flat)
    maxima = _maxabs(tables)

    scale_row = jnp.repeat(maxima[:, 0, 0], EMB).reshape(1, FDIM)
    g2 = gathered.reshape(BATCH, FDIM)
    out2 = _tanh_scale(g2, scale_row)
    return out2.reshape(BATCH, N_CAT, EMB)
